# baseline (speedup 1.0000x reference)
import jax
import jax.numpy as jnp
from jax.experimental import pallas as pl
from jax.experimental.pallas import tpu as pltpu

TOK = 16384
HSD = 2048
CONF = 128
HID2 = 4096
OUT = 1024

BF = jnp.bfloat16
F32 = jnp.float32


def _silu_f32(x):
    return x * jax.nn.sigmoid(x)


def _l1_body(h_ref, u_ref, w1h_ref, w1u_ref, b1_ref, o_ref, hb_ref):
    hb_ref[...] = h_ref[...].astype(BF)
    acc = jnp.dot(hb_ref[...], w1h_ref[...], preferred_element_type=F32)
    acc += jnp.dot(u_ref[...].astype(BF), w1u_ref[...],
                   preferred_element_type=F32)
    acc += b1_ref[...]
    o_ref[...] = _silu_f32(acc).astype(BF)


def _layer1(h, u, w1h, w1u, b1r, tm=512):
    grid = (TOK // tm,)
    return pl.pallas_call(
        _l1_body,
        grid=grid,
        in_specs=[
            pl.BlockSpec((tm, HSD), lambda m: (m, 0)),
            pl.BlockSpec((tm, CONF), lambda m: (m, 0)),
            pl.BlockSpec((HSD, HID2), lambda m: (0, 0)),
            pl.BlockSpec((CONF, HID2), lambda m: (0, 0)),
            pl.BlockSpec((1, HID2), lambda m: (0, 0)),
        ],
        out_specs=pl.BlockSpec((tm, HID2), lambda m: (m, 0)),
        out_shape=jax.ShapeDtypeStruct((TOK, HID2), BF),
        scratch_shapes=[pltpu.VMEM((tm, HSD), BF)],
    )(h, u, w1h, w1u, b1r)


def _l23_body(x_ref, w2_ref, b2_ref, g_ref, be_ref, w3_ref, b3_ref,
              o_ref, acc_ref, xn_ref):
    m = pl.program_id(0)
    cur = acc_ref.at[m % 2]
    prv = acc_ref.at[(m + 1) % 2]

    cur[...] = jnp.dot(x_ref[...], w2_ref[...], preferred_element_type=F32)

    prv[...] = _silu_f32(prv[...] + b2_ref[...])
    a = prv[...]
    s1 = jnp.sum(a, axis=-1, keepdims=True)
    s2 = jnp.sum(a * a, axis=-1, keepdims=True)
    mu = s1 * (1.0 / HID2)
    var = s2 * (1.0 / HID2) - mu * mu
    rs = jax.lax.rsqrt(var + 1e-5)
    xn_ref[...] = ((prv[...] - mu) * rs * g_ref[...]
                   + be_ref[...]).astype(BF)
    o_ref[...] = (jnp.dot(xn_ref[...], w3_ref[...],
                          preferred_element_type=F32) + b3_ref[...])


def _layer23(x1, w2, b2r, gr, ber, w3, b3r, tm=256):
    nblk = TOK // tm
    grid = (nblk + 1,)
    return pl.pallas_call(
        _l23_body,
        grid=grid,
        in_specs=[
            pl.BlockSpec((tm, HID2), lambda m: (jnp.minimum(m, nblk - 1), 0)),
            pl.BlockSpec((HID2, HID2), lambda m: (0, 0)),
            pl.BlockSpec((1, HID2), lambda m: (0, 0)),
            pl.BlockSpec((1, HID2), lambda m: (0, 0)),
            pl.BlockSpec((1, HID2), lambda m: (0, 0)),
            pl.BlockSpec((HID2, OUT), lambda m: (0, 0)),
            pl.BlockSpec((1, OUT), lambda m: (0, 0)),
        ],
        out_specs=pl.BlockSpec((tm, OUT),
                               lambda m: (jnp.maximum(m - 1, 0), 0)),
        out_shape=jax.ShapeDtypeStruct((TOK, OUT), F32),
        scratch_shapes=[pltpu.VMEM((2, tm, HID2), F32),
                        pltpu.VMEM((tm, HID2), BF)],
    )(x1, w2, b2r, gr, ber, w3, b3r)


@jax.jit
def _run(h, u, W1, b1, W2, b2, gamma, beta, W3, b3):
    w1h = W1[:HSD].astype(BF)
    w1u = W1[HSD:].astype(BF)
    x1 = _layer1(h, u, w1h, w1u, b1.reshape(1, -1))
    out = _layer23(x1, W2.astype(BF), b2.reshape(1, -1),
                   gamma.reshape(1, -1), beta.reshape(1, -1),
                   W3.astype(BF), b3.reshape(1, -1))
    return out


def kernel(h, code_emb, u, W1, b1, W2, b2, gamma, beta, W3, b3):
    out = _run(h, u, W1, b1, W2, b2, gamma, beta, W3, b3)
    zero = jnp.array(0.0, dtype=F32)
    return (out, zero, zero, zero, zero)


# speedup vs baseline: 1.0585x; 1.0585x over previous
import jax
import jax.numpy as jnp
from jax.experimental import pallas as pl
from jax.experimental.pallas import tpu as pltpu

TOK = 16384
HSD = 2048
CONF = 128
HID2 = 4096
OUT = 1024

BF = jnp.bfloat16
F32 = jnp.float32


def _silu_f32(x):
    return x * jax.nn.sigmoid(x)


def _l1_body(h_ref, u_ref, w1h_ref, w1u_ref, b1_ref, o_ref, hb_ref):
    hb_ref[...] = h_ref[...].astype(BF)
    acc = jnp.dot(hb_ref[...], w1h_ref[...], preferred_element_type=F32)
    acc += jnp.dot(u_ref[...].astype(BF), w1u_ref[...],
                   preferred_element_type=F32)
    acc += b1_ref[...]
    o_ref[...] = _silu_f32(acc).astype(BF)


def _layer1(h, u, w1h, w1u, b1r, tm=512):
    grid = (TOK // tm,)
    return pl.pallas_call(
        _l1_body,
        grid=grid,
        in_specs=[
            pl.BlockSpec((tm, HSD), lambda m: (m, 0)),
            pl.BlockSpec((tm, CONF), lambda m: (m, 0)),
            pl.BlockSpec((HSD, HID2), lambda m: (0, 0)),
            pl.BlockSpec((CONF, HID2), lambda m: (0, 0)),
            pl.BlockSpec((1, HID2), lambda m: (0, 0)),
        ],
        out_specs=pl.BlockSpec((tm, HID2), lambda m: (m, 0)),
        out_shape=jax.ShapeDtypeStruct((TOK, HID2), BF),
        scratch_shapes=[pltpu.VMEM((tm, HSD), BF)],
    )(h, u, w1h, w1u, b1r)


def _l23_body(x_ref, w2_ref, b2_ref, w3g_ref, vg_ref, c0_ref,
              o_ref, sx_ref):
    sx_ref[...] = _silu_f32(
        jnp.dot(x_ref[...], w2_ref[...], preferred_element_type=F32)
        + b2_ref[...]).astype(BF)
    a = sx_ref[...].astype(F32)
    s1 = jnp.sum(a, axis=-1, keepdims=True)
    s2 = jnp.sum(a * a, axis=-1, keepdims=True)
    mu = s1 * (1.0 / HID2)
    var = s2 * (1.0 / HID2) - mu * mu
    rs = jax.lax.rsqrt(var + 1e-5)
    y = jnp.dot(sx_ref[...], w3g_ref[...], preferred_element_type=F32)
    o_ref[...] = y * rs - (rs * mu) * vg_ref[...] + c0_ref[...]


def _layer23(x1, w2, b2r, w3g, vg, c0, tm=256):
    grid = (TOK // tm,)
    return pl.pallas_call(
        _l23_body,
        grid=grid,
        in_specs=[
            pl.BlockSpec((tm, HID2), lambda m: (m, 0)),
            pl.BlockSpec((HID2, HID2), lambda m: (0, 0)),
            pl.BlockSpec((1, HID2), lambda m: (0, 0)),
            pl.BlockSpec((HID2, OUT), lambda m: (0, 0)),
            pl.BlockSpec((1, OUT), lambda m: (0, 0)),
            pl.BlockSpec((1, OUT), lambda m: (0, 0)),
        ],
        out_specs=pl.BlockSpec((tm, OUT), lambda m: (m, 0)),
        out_shape=jax.ShapeDtypeStruct((TOK, OUT), F32),
        scratch_shapes=[pltpu.VMEM((tm, HID2), BF)],
    )(x1, w2, b2r, w3g, vg, c0)


@jax.jit
def _run(h, u, W1, b1, W2, b2, gamma, beta, W3, b3):
    w1h = W1[:HSD].astype(BF)
    w1u = W1[HSD:].astype(BF)
    x1 = _layer1(h, u, w1h, w1u, b1.reshape(1, -1))
    w3g = (gamma[:, None] * W3).astype(BF)
    hi = jax.lax.Precision.HIGHEST
    vg = jnp.dot(gamma[None, :], W3, precision=hi)
    c0 = jnp.dot(beta[None, :], W3, precision=hi) + b3[None, :]
    out = _layer23(x1, W2.astype(BF), b2.reshape(1, -1), w3g, vg, c0)
    return out


def kernel(h, code_emb, u, W1, b1, W2, b2, gamma, beta, W3, b3):
    out = _run(h, u, W1, b1, W2, b2, gamma, beta, W3, b3)
    zero = jnp.array(0.0, dtype=F32)
    return (out, zero, zero, zero, zero)


# speedup vs baseline: 1.0714x; 1.0122x over previous
import jax
import jax.numpy as jnp
from jax.experimental import pallas as pl
from jax.experimental.pallas import tpu as pltpu

TOK = 16384
HSD = 2048
CONF = 128
HID2 = 4096
OUT = 1024

BF = jnp.bfloat16
F32 = jnp.float32


def _silu_f32(x):
    return x * jax.nn.sigmoid(x)


def _l1_body(h_ref, u_ref, w1h_ref, w1u_ref, b1_ref, o_ref, hb_ref):
    hb_ref[...] = h_ref[...].astype(BF)
    acc = jnp.dot(hb_ref[...], w1h_ref[...], preferred_element_type=F32)
    acc += jnp.dot(u_ref[...].astype(BF), w1u_ref[...],
                   preferred_element_type=F32)
    acc += b1_ref[...]
    o_ref[...] = _silu_f32(acc).astype(BF)


def _layer1(h, u, w1h, w1u, b1r, tm=512):
    grid = (TOK // tm,)
    return pl.pallas_call(
        _l1_body,
        grid=grid,
        in_specs=[
            pl.BlockSpec((tm, HSD), lambda m: (m, 0)),
            pl.BlockSpec((tm, CONF), lambda m: (m, 0)),
            pl.BlockSpec((HSD, HID2), lambda m: (0, 0)),
            pl.BlockSpec((CONF, HID2), lambda m: (0, 0)),
            pl.BlockSpec((1, HID2), lambda m: (0, 0)),
        ],
        out_specs=pl.BlockSpec((tm, HID2), lambda m: (m, 0)),
        out_shape=jax.ShapeDtypeStruct((TOK, HID2), BF),
        scratch_shapes=[pltpu.VMEM((tm, HSD), BF)],
    )(h, u, w1h, w1u, b1r)


def _l23_body(x_ref, w2_ref, b2_ref, w3g_ref, vg_ref, c0_ref,
              o_ref, sx_ref):
    sx_ref[...] = _silu_f32(
        jnp.dot(x_ref[...], w2_ref[...], preferred_element_type=F32)
        + b2_ref[...]).astype(BF)
    a = sx_ref[...].astype(F32)
    s1 = jnp.sum(a, axis=-1, keepdims=True)
    s2 = jnp.sum(a * a, axis=-1, keepdims=True)
    mu = s1 * (1.0 / HID2)
    var = s2 * (1.0 / HID2) - mu * mu
    rs = jax.lax.rsqrt(var + 1e-5)
    y = jnp.dot(sx_ref[...], w3g_ref[...], preferred_element_type=F32)
    o_ref[...] = y * rs - (rs * mu) * vg_ref[...] + c0_ref[...]


def _layer23(x1, w2, b2r, w3g, vg, c0, tm=512):
    grid = (TOK // tm,)
    return pl.pallas_call(
        _l23_body,
        grid=grid,
        in_specs=[
            pl.BlockSpec((tm, HID2), lambda m: (m, 0)),
            pl.BlockSpec((HID2, HID2), lambda m: (0, 0)),
            pl.BlockSpec((1, HID2), lambda m: (0, 0)),
            pl.BlockSpec((HID2, OUT), lambda m: (0, 0)),
            pl.BlockSpec((1, OUT), lambda m: (0, 0)),
            pl.BlockSpec((1, OUT), lambda m: (0, 0)),
        ],
        out_specs=pl.BlockSpec((tm, OUT), lambda m: (m, 0)),
        out_shape=jax.ShapeDtypeStruct((TOK, OUT), F32),
        scratch_shapes=[pltpu.VMEM((tm, HID2), BF)],
        compiler_params=pltpu.CompilerParams(
            vmem_limit_bytes=64 * 1024 * 1024),
    )(x1, w2, b2r, w3g, vg, c0)


@jax.jit
def _run(h, u, W1, b1, W2, b2, gamma, beta, W3, b3):
    w1h = W1[:HSD].astype(BF)
    w1u = W1[HSD:].astype(BF)
    x1 = _layer1(h, u, w1h, w1u, b1.reshape(1, -1))
    w3g = (gamma[:, None] * W3).astype(BF)
    hi = jax.lax.Precision.HIGHEST
    vg = jnp.dot(gamma[None, :], W3, precision=hi)
    c0 = jnp.dot(beta[None, :], W3, precision=hi) + b3[None, :]
    out = _layer23(x1, W2.astype(BF), b2.reshape(1, -1), w3g, vg, c0)
    return out


def kernel(h, code_emb, u, W1, b1, W2, b2, gamma, beta, W3, b3):
    out = _run(h, u, W1, b1, W2, b2, gamma, beta, W3, b3)
    zero = jnp.array(0.0, dtype=F32)
    return (out, zero, zero, zero, zero)
